# baseline (device time: 40405 ns/iter reference)
import jax
import jax.numpy as jnp
from jax import lax
from jax.experimental import pallas as pl
from jax.experimental.pallas import tpu as pltpu

B, S, D = 2, 256, 1024
H, Dh, Dr = 16, 64, 32
DC_HALF = 64
BS = B * S
SCALE = (Dh + Dr) ** -0.5


def kernel(x, Wdkv, Wuk, Wuv, Wq, Wqr, Wkr, Wo):
    def body(x_ref, wdkv_ref, wuk_ref, wuv_ref, wq_ref, wqr_ref, wkr_ref,
             wo_ref, out_ref,
             c_send, c_recv, wk_send, wk_recv, wv_send, wv_recv,
             send_sems, recv_sems):
        my_x = lax.axis_index("x")
        my_y = lax.axis_index("y")
        my_z = lax.axis_index("z")
        peer = (my_x, 1 - my_y, my_z)

        barrier_sem = pltpu.get_barrier_semaphore()
        pl.semaphore_signal(barrier_sem, inc=1, device_id=peer,
                            device_id_type=pl.DeviceIdType.MESH)
        pl.semaphore_wait(barrier_sem, 1)

        xb = x_ref[...].reshape(BS, D).astype(jnp.bfloat16)

        c_send[...] = jnp.dot(
            xb, wdkv_ref[...].astype(jnp.bfloat16),
            preferred_element_type=jnp.float32).astype(jnp.bfloat16)
        wk_send[...] = wuk_ref[...].astype(jnp.bfloat16)
        wv_send[...] = wuv_ref[...].astype(jnp.bfloat16)

        copies = []
        for i, (src, dst) in enumerate(
                [(c_send, c_recv), (wk_send, wk_recv), (wv_send, wv_recv)]):
            rdma = pltpu.make_async_remote_copy(
                src_ref=src, dst_ref=dst,
                send_sem=send_sems.at[i], recv_sem=recv_sems.at[i],
                device_id=peer, device_id_type=pl.DeviceIdType.MESH)
            rdma.start()
            copies.append(rdma)

        Q = jnp.dot(xb, wq_ref[...].astype(jnp.bfloat16),
                    preferred_element_type=jnp.float32).astype(jnp.bfloat16)
        Qr = jnp.dot(xb, wqr_ref[...].astype(jnp.bfloat16),
                     preferred_element_type=jnp.float32).astype(jnp.bfloat16)
        Kr = jnp.dot(xb, wkr_ref[...].astype(jnp.bfloat16),
                     preferred_element_type=jnp.float32).astype(jnp.bfloat16)

        for rdma in copies:
            rdma.wait()

        cm = c_send[...]
        cp = c_recv[...]
        K = (jnp.dot(cm, wk_send[...], preferred_element_type=jnp.float32)
             + jnp.dot(cp, wk_recv[...], preferred_element_type=jnp.float32)
             ).astype(jnp.bfloat16)
        V = (jnp.dot(cm, wv_send[...], preferred_element_type=jnp.float32)
             + jnp.dot(cp, wv_recv[...], preferred_element_type=jnp.float32)
             ).astype(jnp.bfloat16)

        contract_last = (((1,), (1,)), ((), ()))
        for b in range(B):
            row = slice(b * S, (b + 1) * S)
            kr_b = Kr[row, :]
            head_outs = []
            for h in range(H):
                col = slice(h * Dh, (h + 1) * Dh)
                rcol = slice(h * Dr, (h + 1) * Dr)
                q = Q[row, col]
                qr = Qr[row, rcol]
                k = K[row, col]
                v = V[row, col]
                scores = (
                    lax.dot_general(q, k, contract_last,
                                    preferred_element_type=jnp.float32)
                    + lax.dot_general(qr, kr_b, contract_last,
                                      preferred_element_type=jnp.float32)
                ) * SCALE
                m = jnp.max(scores, axis=-1, keepdims=True)
                p = jnp.exp(scores - m)
                p = (p / jnp.sum(p, axis=-1, keepdims=True)).astype(jnp.bfloat16)
                head_outs.append(
                    jnp.dot(p, v, preferred_element_type=jnp.float32)
                    .astype(jnp.bfloat16))
            o_b = jnp.concatenate(head_outs, axis=1)
            out_ref[b, :, :] = jnp.dot(
                o_b, wo_ref[...].astype(jnp.bfloat16),
                preferred_element_type=jnp.float32)

    return pl.pallas_call(
        body,
        out_shape=jax.ShapeDtypeStruct((B, S, D), jnp.float32),
        in_specs=[pl.BlockSpec(memory_space=pltpu.VMEM)] * 8,
        out_specs=pl.BlockSpec(memory_space=pltpu.VMEM),
        scratch_shapes=[
            pltpu.VMEM((BS, DC_HALF), jnp.bfloat16),
            pltpu.VMEM((BS, DC_HALF), jnp.bfloat16),
            pltpu.VMEM((DC_HALF, D), jnp.bfloat16),
            pltpu.VMEM((DC_HALF, D), jnp.bfloat16),
            pltpu.VMEM((DC_HALF, D), jnp.bfloat16),
            pltpu.VMEM((DC_HALF, D), jnp.bfloat16),
            pltpu.SemaphoreType.DMA((3,)),
            pltpu.SemaphoreType.DMA((3,)),
        ],
        compiler_params=pltpu.CompilerParams(collective_id=0),
    )(x, Wdkv, Wuk, Wuv, Wq, Wqr, Wkr, Wo)


# device time: 33284 ns/iter; 1.2139x vs baseline; 1.2139x over previous
import jax
import jax.numpy as jnp
from jax import lax
from jax.experimental import pallas as pl
from jax.experimental.pallas import tpu as pltpu

B, S, D = 2, 256, 1024
H, Dh, Dr = 16, 64, 32
DC_HALF = 64
BS = B * S
SCALE = (Dh + Dr) ** -0.5


def kernel(x, Wdkv, Wuk, Wuv, Wq, Wqr, Wkr, Wo):
    def body(x_ref, wdkv_ref, wuk_ref, wuv_ref, wq_ref, wqr_ref, wkr_ref,
             wo_ref, out_ref,
             c_send, c_recv, wk_send, wk_recv, wv_send, wv_recv,
             send_sems, recv_sems):
        my_x = lax.axis_index("x")
        my_y = lax.axis_index("y")
        my_z = lax.axis_index("z")
        peer = (my_x, 1 - my_y, my_z)

        barrier_sem = pltpu.get_barrier_semaphore()
        pl.semaphore_signal(barrier_sem, inc=1, device_id=peer,
                            device_id_type=pl.DeviceIdType.MESH)
        pl.semaphore_wait(barrier_sem, 1)

        xb = x_ref[...].reshape(BS, D).astype(jnp.bfloat16)

        c_send[...] = jnp.dot(
            xb, wdkv_ref[...].astype(jnp.bfloat16),
            preferred_element_type=jnp.float32).astype(jnp.bfloat16)
        wk_send[...] = wuk_ref[...].astype(jnp.bfloat16)
        wv_send[...] = wuv_ref[...].astype(jnp.bfloat16)

        copies = []
        for i, (src, dst) in enumerate(
                [(c_send, c_recv), (wk_send, wk_recv), (wv_send, wv_recv)]):
            rdma = pltpu.make_async_remote_copy(
                src_ref=src, dst_ref=dst,
                send_sem=send_sems.at[i], recv_sem=recv_sems.at[i],
                device_id=peer, device_id_type=pl.DeviceIdType.MESH)
            rdma.start()
            copies.append(rdma)

        Q = (jnp.dot(xb, wq_ref[...].astype(jnp.bfloat16),
                     preferred_element_type=jnp.float32)
             * SCALE).astype(jnp.bfloat16)
        Qr = (jnp.dot(xb, wqr_ref[...].astype(jnp.bfloat16),
                      preferred_element_type=jnp.float32)
              * SCALE).astype(jnp.bfloat16)
        Kr = jnp.dot(xb, wkr_ref[...].astype(jnp.bfloat16),
                     preferred_element_type=jnp.float32).astype(jnp.bfloat16)

        for rdma in copies:
            rdma.wait()

        cm = c_send[...]
        cp = c_recv[...]
        K = (jnp.dot(cm, wk_send[...], preferred_element_type=jnp.float32)
             + jnp.dot(cp, wk_recv[...], preferred_element_type=jnp.float32)
             ).astype(jnp.bfloat16)
        V = (jnp.dot(cm, wv_send[...], preferred_element_type=jnp.float32)
             + jnp.dot(cp, wv_recv[...], preferred_element_type=jnp.float32)
             ).astype(jnp.bfloat16)

        contract_last = (((1,), (1,)), ((), ()))
        for b in range(B):
            row = slice(b * S, (b + 1) * S)
            kr_b = Kr[row, :]
            head_outs = []
            for h in range(H):
                col = slice(h * Dh, (h + 1) * Dh)
                rcol = slice(h * Dr, (h + 1) * Dr)
                q = Q[row, col]
                qr = Qr[row, rcol]
                k = K[row, col]
                v = V[row, col]
                scores = (
                    lax.dot_general(q, k, contract_last,
                                    preferred_element_type=jnp.float32)
                    + lax.dot_general(qr, kr_b, contract_last,
                                      preferred_element_type=jnp.float32)
                )
                p = jnp.exp(scores)
                r = 1.0 / jnp.sum(p, axis=-1, keepdims=True)
                o = jnp.dot(p.astype(jnp.bfloat16), v,
                            preferred_element_type=jnp.float32)
                head_outs.append((o * r).astype(jnp.bfloat16))
            o_b = jnp.concatenate(head_outs, axis=1)
            out_ref[b, :, :] = jnp.dot(
                o_b, wo_ref[...].astype(jnp.bfloat16),
                preferred_element_type=jnp.float32)

    return pl.pallas_call(
        body,
        out_shape=jax.ShapeDtypeStruct((B, S, D), jnp.float32),
        in_specs=[pl.BlockSpec(memory_space=pltpu.VMEM)] * 8,
        out_specs=pl.BlockSpec(memory_space=pltpu.VMEM),
        scratch_shapes=[
            pltpu.VMEM((BS, DC_HALF), jnp.bfloat16),
            pltpu.VMEM((BS, DC_HALF), jnp.bfloat16),
            pltpu.VMEM((DC_HALF, D), jnp.bfloat16),
            pltpu.VMEM((DC_HALF, D), jnp.bfloat16),
            pltpu.VMEM((DC_HALF, D), jnp.bfloat16),
            pltpu.VMEM((DC_HALF, D), jnp.bfloat16),
            pltpu.SemaphoreType.DMA((3,)),
            pltpu.SemaphoreType.DMA((3,)),
        ],
        compiler_params=pltpu.CompilerParams(collective_id=0),
    )(x, Wdkv, Wuk, Wuv, Wq, Wqr, Wkr, Wo)


# device time: 26619 ns/iter; 1.5179x vs baseline; 1.2504x over previous
import jax
import jax.numpy as jnp
from jax import lax
from jax.experimental import pallas as pl
from jax.experimental.pallas import tpu as pltpu

B, S, D = 2, 256, 1024
H, Dh, Dr = 16, 64, 32
DC_HALF = 64
BS = B * S
SCALE = (Dh + Dr) ** -0.5
NQQR = H * Dh + H * Dr
NPROJ = NQQR + Dr

_AtB = (((0,), (0,)), ((), ()))


def kernel(x, Wdkv, Wuk, Wuv, Wq, Wqr, Wkr, Wo):
    def body(x_ref, wdkv_ref, wuk_ref, wuv_ref, wq_ref, wqr_ref, wkr_ref,
             wo_ref, out_ref,
             c_send, c_recv, w_send, w_recv, proj_w,
             send_sems, recv_sems):
        my_x = lax.axis_index("x")
        my_y = lax.axis_index("y")
        my_z = lax.axis_index("z")
        peer = (my_x, 1 - my_y, my_z)

        barrier_sem = pltpu.get_barrier_semaphore()
        pl.semaphore_signal(barrier_sem, inc=1, device_id=peer,
                            device_id_type=pl.DeviceIdType.MESH)
        pl.semaphore_wait(barrier_sem, 1)

        def start_copy(i, src, dst):
            rdma = pltpu.make_async_remote_copy(
                src_ref=src, dst_ref=dst,
                send_sem=send_sems.at[i], recv_sem=recv_sems.at[i],
                device_id=peer, device_id_type=pl.DeviceIdType.MESH)
            rdma.start()
            return rdma

        w_send[:, :D] = wuk_ref[...].astype(jnp.bfloat16)
        w_send[:, D:] = wuv_ref[...].astype(jnp.bfloat16)
        rd_w = start_copy(1, w_send, w_recv)

        xt = jnp.transpose(
            x_ref[...].reshape(BS, D).astype(jnp.bfloat16))

        c_send[...] = lax.dot_general(
            wdkv_ref[...].astype(jnp.bfloat16), xt, _AtB,
            preferred_element_type=jnp.float32).astype(jnp.bfloat16)
        rd_c = start_copy(0, c_send, c_recv)

        proj_w[:, :H * Dh] = wq_ref[...].astype(jnp.bfloat16)
        proj_w[:, H * Dh:NQQR] = wqr_ref[...].astype(jnp.bfloat16)
        proj_w[:, NQQR:] = wkr_ref[...].astype(jnp.bfloat16)
        P = lax.dot_general(proj_w[...], xt, _AtB,
                            preferred_element_type=jnp.float32)
        Qt = (P[:H * Dh] * SCALE).astype(jnp.bfloat16)
        Qrt = (P[H * Dh:NQQR] * SCALE).astype(jnp.bfloat16)
        Krt = P[NQQR:].astype(jnp.bfloat16)

        rd_c.wait()
        rd_w.wait()

        C2 = jnp.concatenate([c_send[...], c_recv[...]], axis=0)
        W2 = jnp.concatenate([w_send[...], w_recv[...]], axis=0)
        KV = lax.dot_general(W2, C2, _AtB,
                             preferred_element_type=jnp.float32)
        Kt = KV[:D].astype(jnp.bfloat16)
        Vt = KV[D:].astype(jnp.bfloat16)

        _bdot_c1 = (((1,), (1,)), ((0,), (0,)))
        _bdot_c2 = (((2,), (2,)), ((0,), (0,)))
        batch_outs = []
        for b in range(B):
            col = slice(b * S, (b + 1) * S)
            q4 = Qt[:, col].reshape(H, Dh, S)
            k4 = Kt[:, col].reshape(H, Dh, S)
            v4 = Vt[:, col].reshape(H, Dh, S)
            qr4 = Qrt[:, col].reshape(H, Dr, S)
            kr4 = jnp.broadcast_to(Krt[:, col][None], (H, Dr, S))
            scores = (
                lax.dot_general(q4, k4, _bdot_c1,
                                preferred_element_type=jnp.float32)
                + lax.dot_general(qr4, kr4, _bdot_c1,
                                  preferred_element_type=jnp.float32)
            )
            p = jnp.exp(scores)
            r = 1.0 / jnp.sum(p, axis=2)
            o = lax.dot_general(v4, p.astype(jnp.bfloat16), _bdot_c2,
                                preferred_element_type=jnp.float32)
            o = o * r[:, None, :]
            batch_outs.append(o.reshape(H * Dh, S).astype(jnp.bfloat16))
        Ot = jnp.concatenate(batch_outs, axis=1)

        out2d = lax.dot_general(Ot, wo_ref[...].astype(jnp.bfloat16), _AtB,
                                preferred_element_type=jnp.float32)
        out_ref[...] = out2d.reshape(B, S, D)

    return pl.pallas_call(
        body,
        out_shape=jax.ShapeDtypeStruct((B, S, D), jnp.float32),
        in_specs=[pl.BlockSpec(memory_space=pltpu.VMEM)] * 8,
        out_specs=pl.BlockSpec(memory_space=pltpu.VMEM),
        scratch_shapes=[
            pltpu.VMEM((DC_HALF, BS), jnp.bfloat16),
            pltpu.VMEM((DC_HALF, BS), jnp.bfloat16),
            pltpu.VMEM((DC_HALF, 2 * D), jnp.bfloat16),
            pltpu.VMEM((DC_HALF, 2 * D), jnp.bfloat16),
            pltpu.VMEM((D, NPROJ), jnp.bfloat16),
            pltpu.SemaphoreType.DMA((2,)),
            pltpu.SemaphoreType.DMA((2,)),
        ],
        compiler_params=pltpu.CompilerParams(collective_id=0),
    )(x, Wdkv, Wuk, Wuv, Wq, Wqr, Wkr, Wo)


# device time: 25530 ns/iter; 1.5826x vs baseline; 1.0427x over previous
import jax
import jax.numpy as jnp
from jax import lax
from jax.experimental import pallas as pl
from jax.experimental.pallas import tpu as pltpu

B, S, D = 2, 256, 1024
H, Dh, Dr = 16, 64, 32
DC_HALF = 64
BS = B * S
SCALE = (Dh + Dr) ** -0.5
NQQR = H * Dh + H * Dr
NPROJ = NQQR + Dr

_AtB = (((0,), (0,)), ((), ()))


def kernel(x, Wdkv, Wuk, Wuv, Wq, Wqr, Wkr, Wo):
    def body(x_ref, wdkv_ref, wuk_ref, wuv_ref, wq_ref, wqr_ref, wkr_ref,
             wo_ref, out_ref,
             c_send, c_recv, w_send, w_recv, proj_w,
             send_sems, recv_sems):
        my_x = lax.axis_index("x")
        my_y = lax.axis_index("y")
        my_z = lax.axis_index("z")
        peer = (my_x, 1 - my_y, my_z)

        barrier_sem = pltpu.get_barrier_semaphore()
        pl.semaphore_signal(barrier_sem, inc=1, device_id=peer,
                            device_id_type=pl.DeviceIdType.MESH)
        pl.semaphore_wait(barrier_sem, 1)

        def start_copy(i, src, dst):
            rdma = pltpu.make_async_remote_copy(
                src_ref=src, dst_ref=dst,
                send_sem=send_sems.at[i], recv_sem=recv_sems.at[i],
                device_id=peer, device_id_type=pl.DeviceIdType.MESH)
            rdma.start()
            return rdma

        w_send[:, :D] = wuk_ref[...].astype(jnp.bfloat16)
        w_send[:, D:] = wuv_ref[...].astype(jnp.bfloat16)
        rd_w = start_copy(1, w_send, w_recv)

        xt = jnp.transpose(
            x_ref[...].reshape(BS, D).astype(jnp.bfloat16))

        c_send[...] = lax.dot_general(
            wdkv_ref[...].astype(jnp.bfloat16), xt, _AtB,
            preferred_element_type=jnp.float32).astype(jnp.bfloat16)
        rd_c = start_copy(0, c_send, c_recv)

        proj_w[:, :H * Dh] = wq_ref[...].astype(jnp.bfloat16)
        proj_w[:, H * Dh:NQQR] = wqr_ref[...].astype(jnp.bfloat16)
        proj_w[:, NQQR:] = wkr_ref[...].astype(jnp.bfloat16)
        P = lax.dot_general(proj_w[...], xt, _AtB,
                            preferred_element_type=jnp.float32)
        Qt = (P[:H * Dh] * SCALE).astype(jnp.bfloat16)
        Qrt = (P[H * Dh:NQQR] * SCALE).astype(jnp.bfloat16)
        Krt = P[NQQR:].astype(jnp.bfloat16)

        _bdot_c1 = (((1,), (1,)), ((0,), (0,)))
        scoresR = []
        for b in range(B):
            col = slice(b * S, (b + 1) * S)
            qr4 = Qrt[:, col].reshape(H, Dr, S)
            kr4 = jnp.broadcast_to(Krt[:, col][None], (H, Dr, S))
            scoresR.append(
                lax.dot_general(qr4, kr4, _bdot_c1,
                                preferred_element_type=jnp.float32))

        rd_c.wait()
        rd_w.wait()

        C2 = jnp.concatenate([c_send[...], c_recv[...]], axis=0)
        W2 = jnp.concatenate([w_send[...], w_recv[...]], axis=0)
        KV = lax.dot_general(W2, C2, _AtB,
                             preferred_element_type=jnp.float32)
        Kt = KV[:D].astype(jnp.bfloat16)
        Vt = KV[D:].astype(jnp.bfloat16)

        _bdot_c2 = (((2,), (2,)), ((0,), (0,)))
        batch_outs = []
        for b in range(B):
            col = slice(b * S, (b + 1) * S)
            q4 = Qt[:, col].reshape(H, Dh, S)
            k4 = Kt[:, col].reshape(H, Dh, S)
            v4 = Vt[:, col].reshape(H, Dh, S)
            scores = (
                lax.dot_general(q4, k4, _bdot_c1,
                                preferred_element_type=jnp.float32)
                + scoresR[b]
            )
            p = jnp.exp(scores)
            r = 1.0 / jnp.sum(p, axis=2)
            o = lax.dot_general(v4, p.astype(jnp.bfloat16), _bdot_c2,
                                preferred_element_type=jnp.float32)
            o = o * r[:, None, :]
            batch_outs.append(o.reshape(H * Dh, S).astype(jnp.bfloat16))
        Ot = jnp.concatenate(batch_outs, axis=1)

        out2d = lax.dot_general(Ot, wo_ref[...].astype(jnp.bfloat16), _AtB,
                                preferred_element_type=jnp.float32)
        out_ref[...] = out2d.reshape(B, S, D)

    return pl.pallas_call(
        body,
        out_shape=jax.ShapeDtypeStruct((B, S, D), jnp.float32),
        in_specs=[pl.BlockSpec(memory_space=pltpu.VMEM)] * 8,
        out_specs=pl.BlockSpec(memory_space=pltpu.VMEM),
        scratch_shapes=[
            pltpu.VMEM((DC_HALF, BS), jnp.bfloat16),
            pltpu.VMEM((DC_HALF, BS), jnp.bfloat16),
            pltpu.VMEM((DC_HALF, 2 * D), jnp.bfloat16),
            pltpu.VMEM((DC_HALF, 2 * D), jnp.bfloat16),
            pltpu.VMEM((D, NPROJ), jnp.bfloat16),
            pltpu.SemaphoreType.DMA((2,)),
            pltpu.SemaphoreType.DMA((2,)),
        ],
        compiler_params=pltpu.CompilerParams(collective_id=0),
    )(x, Wdkv, Wuk, Wuv, Wq, Wqr, Wkr, Wo)
